# Initial kernel scaffold; baseline (speedup 1.0000x reference)
import functools

import jax
import jax.numpy as jnp
from jax import lax
from jax.experimental import pallas as pl
from jax.experimental.pallas import tpu as pltpu
from jax.experimental.pallas import tpu_sc as plsc

N = 10000
D = 128
E = 320000

K = 128
NC = 2
NS = 16
NW = NC * NS
CHUNKS = -(-E // (NW * K))
EPT = CHUNKS * K
E_PAD = NW * EPT

ROWS_PER_TILE = N // NS
OUT_CHUNK = 125


def _sc_aggregate(x, src, dst, ew):
    mesh = plsc.VectorSubcoreMesh(core_axis_name="c", subcore_axis_name="s")

    @functools.partial(
        pl.kernel,
        out_type=jax.ShapeDtypeStruct((NC, N, D), jnp.float32),
        mesh=mesh,
        scratch_types=[
            pltpu.VMEM((K,), jnp.int32),
            pltpu.VMEM((K,), jnp.int32),
            pltpu.VMEM((K,), jnp.float32),
            pltpu.VMEM((K, D), jnp.float32),
            pltpu.VMEM((OUT_CHUNK, D), jnp.float32),
            pltpu.VMEM_SHARED((N, D), jnp.float32),
            pltpu.SemaphoreType.DMA,
        ],
    )
    def agg(x_hbm, src_hbm, dst_hbm, ew_hbm, part_hbm,
            sidx_v, didx_v, ew_v, rows_v, obuf_v, acc_sh, sem):
        cid = lax.axis_index("c")
        sid = lax.axis_index("s")
        wid = cid * NS + sid

        zero16 = jnp.zeros((16,), jnp.float32)

        @pl.loop(0, OUT_CHUNK)
        def _zero_rows(r):
            for c in range(D // 16):
                obuf_v[r, pl.ds(c * 16, 16)] = zero16

        @pl.loop(0, ROWS_PER_TILE // OUT_CHUNK)
        def _zero_acc(t):
            pltpu.sync_copy(
                obuf_v, acc_sh.at[pl.ds(sid * ROWS_PER_TILE + t * OUT_CHUNK,
                                        OUT_CHUNK)])

        plsc.subcore_barrier()

        @pl.loop(0, CHUNKS)
        def _edge_chunk(j):
            base = wid * EPT + j * K
            pltpu.sync_copy(src_hbm.at[pl.ds(base, K)], sidx_v)
            pltpu.sync_copy(dst_hbm.at[pl.ds(base, K)], didx_v)
            pltpu.sync_copy(ew_hbm.at[pl.ds(base, K)], ew_v)
            pltpu.async_copy(x_hbm.at[sidx_v], rows_v, sem).wait()

            @pl.loop(0, K // 16)
            def _scale_group(g):
                for jj in range(16):
                    e = g * 16 + jj
                    w = plsc.load_gather(
                        ew_v, [jnp.zeros((16,), jnp.int32) + e])
                    for c in range(D // 16):
                        rows_v[e, pl.ds(c * 16, 16)] = (
                            rows_v[e, pl.ds(c * 16, 16)] * w)

            pltpu.sync_copy(rows_v, acc_sh.at[didx_v], add=True)

        plsc.subcore_barrier()

        @pl.loop(0, ROWS_PER_TILE // OUT_CHUNK)
        def _copy_out(t):
            row0 = sid * ROWS_PER_TILE + t * OUT_CHUNK
            pltpu.sync_copy(acc_sh.at[pl.ds(row0, OUT_CHUNK)], obuf_v)
            pltpu.sync_copy(obuf_v, part_hbm.at[cid, pl.ds(row0, OUT_CHUNK)])

    return agg(x, src, dst, ew)


def _combine_body(p_ref, w_ref, o_ref):
    o_ref[...] = (p_ref[0] + p_ref[1]) * w_ref[...]


def _tc_combine(part, W):
    blk = 2000
    return pl.pallas_call(
        _combine_body,
        out_shape=jax.ShapeDtypeStruct((N, D), jnp.float32),
        grid=(N // blk,),
        in_specs=[
            pl.BlockSpec((NC, blk, D), lambda i: (0, i, 0)),
            pl.BlockSpec((1, D), lambda i: (0, 0)),
        ],
        out_specs=pl.BlockSpec((blk, D), lambda i: (i, 0)),
    )(part, W.reshape(1, D))


def kernel(x, edge_index, edge_weight, W):
    dst = edge_index[0]
    src = edge_index[1]
    pad = E_PAD - E
    src_p = jnp.concatenate([src, jnp.zeros((pad,), jnp.int32)])
    dst_p = jnp.concatenate([dst, jnp.zeros((pad,), jnp.int32)])
    ew_p = jnp.concatenate([edge_weight, jnp.zeros((pad,), jnp.float32)])
    part = _sc_aggregate(x, src_p, dst_p, ew_p)
    return _tc_combine(part, W)


# speedup vs baseline: 3.7594x; 3.7594x over previous
import functools

import jax
import jax.numpy as jnp
from jax import lax
from jax.experimental import pallas as pl
from jax.experimental.pallas import tpu as pltpu
from jax.experimental.pallas import tpu_sc as plsc

N = 10000
D = 128
E = 320000

K = 128
NC = 2
NS = 16
NW = NC * NS
CHUNKS = -(-E // (NW * K))
EPT = CHUNKS * K
E_PAD = NW * EPT

OUT_CHUNK = 200
N_OUT_CHUNKS = N // OUT_CHUNK


def _sc_aggregate(x, src, dst, ew):
    mesh = plsc.VectorSubcoreMesh(core_axis_name="c", subcore_axis_name="s")

    @functools.partial(
        pl.kernel,
        out_type=jax.ShapeDtypeStruct((NC, N, D), jnp.float32),
        mesh=mesh,
        scratch_types=[
            pltpu.VMEM((K,), jnp.int32),
            pltpu.VMEM((K,), jnp.int32),
            pltpu.VMEM((K,), jnp.float32),
            pltpu.VMEM((K, D), jnp.float32),
            pltpu.VMEM((OUT_CHUNK, D), jnp.float32),
            pltpu.VMEM_SHARED((N, D), jnp.float32),
            pltpu.SemaphoreType.DMA,
        ],
    )
    def agg(x_hbm, src_hbm, dst_hbm, ew_hbm, part_hbm,
            sidx_v, didx_v, ew_v, rows_v, obuf_v, acc_sh, sem):
        cid = lax.axis_index("c")
        sid = lax.axis_index("s")
        wid = cid * NS + sid

        zero16 = jnp.zeros((16,), jnp.float32)

        @pl.loop(0, OUT_CHUNK)
        def _zero_rows(r):
            for c in range(D // 16):
                obuf_v[r, pl.ds(c * 16, 16)] = zero16

        @pl.loop(sid, N_OUT_CHUNKS, step=NS)
        def _zero_acc(t):
            pltpu.sync_copy(obuf_v, acc_sh.at[pl.ds(t * OUT_CHUNK, OUT_CHUNK)])

        plsc.subcore_barrier()

        @pl.loop(0, CHUNKS)
        def _edge_chunk(j):
            base = wid * EPT + j * K
            pltpu.sync_copy(src_hbm.at[pl.ds(base, K)], sidx_v)
            pltpu.sync_copy(dst_hbm.at[pl.ds(base, K)], didx_v)
            pltpu.sync_copy(ew_hbm.at[pl.ds(base, K)], ew_v)
            pltpu.async_copy(x_hbm.at[sidx_v], rows_v, sem).wait()

            @pl.loop(0, K // 16)
            def _scale_group(g):
                ewg = ew_v[pl.ds(g * 16, 16)]
                for jj in range(16):
                    e = g * 16 + jj
                    w = ewg[jj]
                    for c in range(D // 16):
                        rows_v[e, pl.ds(c * 16, 16)] = (
                            rows_v[e, pl.ds(c * 16, 16)] * w)

            pltpu.sync_copy(rows_v, acc_sh.at[didx_v], add=True)

        plsc.subcore_barrier()

        @pl.loop(sid, N_OUT_CHUNKS, step=NS)
        def _copy_out(t):
            row0 = t * OUT_CHUNK
            pltpu.sync_copy(acc_sh.at[pl.ds(row0, OUT_CHUNK)], obuf_v)
            pltpu.sync_copy(obuf_v, part_hbm.at[cid, pl.ds(row0, OUT_CHUNK)])

    return agg(x, src, dst, ew)


def _combine_body(p_ref, w_ref, o_ref):
    o_ref[...] = (p_ref[0] + p_ref[1]) * w_ref[...]


def _tc_combine(part, W):
    blk = 2000
    return pl.pallas_call(
        _combine_body,
        out_shape=jax.ShapeDtypeStruct((N, D), jnp.float32),
        grid=(N // blk,),
        in_specs=[
            pl.BlockSpec((NC, blk, D), lambda i: (0, i, 0)),
            pl.BlockSpec((1, D), lambda i: (0, 0)),
        ],
        out_specs=pl.BlockSpec((blk, D), lambda i: (i, 0)),
    )(part, W.reshape(1, D))


def kernel(x, edge_index, edge_weight, W):
    dst = edge_index[0]
    src = edge_index[1]
    pad = E_PAD - E
    src_p = jnp.concatenate([src, jnp.zeros((pad,), jnp.int32)])
    dst_p = jnp.concatenate([dst, jnp.zeros((pad,), jnp.int32)])
    ew_p = jnp.concatenate([edge_weight, jnp.zeros((pad,), jnp.float32)])
    part = _sc_aggregate(x, src_p, dst_p, ew_p)
    return _tc_combine(part, W)
